# Initial kernel scaffold; baseline (speedup 1.0000x reference)
import jax
import jax.numpy as jnp
from jax.experimental import pallas as pl


def kernel(patch_logits, coords, output_size, prev_pred):
    raise NotImplementedError("write your pallas kernel here")


# speedup vs baseline: 2.6244x; 2.6244x over previous
import functools

import jax
import jax.numpy as jnp
from jax import lax
from jax.experimental import pallas as pl
from jax.experimental.pallas import tpu as pltpu
from jax.experimental.pallas import tpu_sc as plsc

MIN_COV = 1e-6

B, K, C, PS = 2, 1024, 32, 16
H, W = 512, 512
WH = 16
CG = 8
NCG = C // CG
NYW = H // WH
NROUNDS = B * NCG
KP = K + 16
SEW = NYW + 16


def _sc_scatter(patch_logits, packed, se):
  mesh = plsc.VectorSubcoreMesh(core_axis_name="c", subcore_axis_name="s")

  @functools.partial(
      pl.kernel,
      out_type=(
          jax.ShapeDtypeStruct((B, C, H * W), jnp.float32),
          jax.ShapeDtypeStruct((B, H * W), jnp.float32),
      ),
      mesh=mesh,
      compiler_params=pltpu.CompilerParams(needs_layout_passes=False),
      scratch_types=[
          pltpu.VMEM((B * KP,), jnp.int32),
          pltpu.VMEM((B * 2 * SEW,), jnp.int32),
          pltpu.VMEM((CG * WH * W,), jnp.float32),
          pltpu.VMEM((WH * W,), jnp.float32),
          pltpu.VMEM((CG, PS, PS), jnp.float32),
      ],
  )
  def scatter_kernel(patch_hbm, packed_hbm, se_hbm, out_hbm, cnt_hbm,
                     pk_v, se_v, canvas, cntw, buf):
    cid = lax.axis_index("c")
    sid = lax.axis_index("s")
    wid = sid * 2 + cid
    rbase = wid * WH

    pltpu.sync_copy(packed_hbm, pk_v)
    pltpu.sync_copy(se_hbm, se_v)

    zeros16 = jnp.zeros((16,), jnp.float32)
    ones16 = jnp.ones((16,), jnp.float32)
    iota16 = lax.iota(jnp.int32, 16)

    def _scalar_at(ref, flat_idx):
      return plsc.load_gather(ref, [jnp.full((16,), flat_idx, jnp.int32)])[0]

    def round_body(t, _):
      b = t // NCG
      cg = lax.rem(t, NCG)

      def zrow(q, _):
        base = pl.multiple_of(q * 256, 256)
        for j in range(16):
          canvas[pl.ds(base + j * 16, 16)] = zeros16
        return 0
      lax.fori_loop(0, CG * WH * W // 256, zrow, 0)

      def zcnt(q, _):
        base = pl.multiple_of(q * 256, 256)
        for j in range(16):
          cntw[pl.ds(base + j * 16, 16)] = zeros16
        return 0
      lax.fori_loop(0, WH * W // 256, zcnt, 0)

      s = _scalar_at(se_v, (b * 2 + 0) * SEW + wid)
      e = _scalar_at(se_v, (b * 2 + 1) * SEW + wid)

      def pbody(i, _):
        p = _scalar_at(pk_v, b * KP + i)
        k = lax.shift_right_logical(p, 18)
        r = lax.shift_right_logical(p, 9) & 511
        cc = p & 511
        pltpu.sync_copy(patch_hbm.at[b, k, pl.ds(cg * CG, CG)], buf)
        xidx = cc + iota16
        for dy in range(PS):
          yl = r + dy - rbase
          ok = (yl >= 0) & (yl < WH)

          @pl.when(ok)
          def _add():
            idx0 = yl * W + xidx
            for ch in range(CG):
              plsc.addupdate_scatter(canvas, [idx0 + ch * (WH * W)],
                                     buf[ch, dy, :])

            @pl.when(cg == 0)
            def _cnt():
              plsc.addupdate_scatter(cntw, [idx0], ones16)

        return 0

      lax.fori_loop(s, e, pbody, 0)

      for ch in range(CG):
        pltpu.sync_copy(
            canvas.at[pl.ds(ch * WH * W, WH * W)],
            out_hbm.at[b, cg * CG + ch, pl.ds(rbase * W, WH * W)])

      @pl.when(cg == 0)
      def _flush_cnt():
        pltpu.sync_copy(cntw, cnt_hbm.at[b, pl.ds(rbase * W, WH * W)])

      return 0

    lax.fori_loop(0, NROUNDS, round_body, 0)

  return scatter_kernel(patch_logits, packed, se)


def _norm_body(can_ref, cnt_ref, prev_ref, out_ref):
  cnt = cnt_ref[...]
  covered = cnt > MIN_COV
  safe = jnp.maximum(cnt, MIN_COV)
  out_ref[...] = jnp.where(covered[:, None],
                           can_ref[...] / safe[:, None],
                           prev_ref[...])


def _normalize(canvas, counts, prev_pred):
  grid = (B, NCG)
  return pl.pallas_call(
      _norm_body,
      grid=grid,
      in_specs=[
          pl.BlockSpec((1, CG, H, W), lambda b, g: (b, g, 0, 0)),
          pl.BlockSpec((1, H, W), lambda b, g: (b, 0, 0)),
          pl.BlockSpec((1, CG, H, W), lambda b, g: (b, g, 0, 0)),
      ],
      out_specs=pl.BlockSpec((1, CG, H, W), lambda b, g: (b, g, 0, 0)),
      out_shape=jax.ShapeDtypeStruct((B, C, H, W), jnp.float32),
  )(canvas, counts, prev_pred)


def kernel(patch_logits, coords, output_size, prev_pred):
  del output_size
  r = coords[:, :, 0].astype(jnp.int32)
  cc = coords[:, :, 1].astype(jnp.int32)
  order = jnp.argsort(r, axis=1).astype(jnp.int32)
  r_s = jnp.take_along_axis(r, order, axis=1)
  c_s = jnp.take_along_axis(cc, order, axis=1)
  packed = (order << 18) | (r_s << 9) | c_s

  rv = jnp.arange(NYW, dtype=jnp.int32) * WH
  starts = jax.vmap(lambda rs: jnp.searchsorted(rs, rv - (PS - 1)))(r_s)
  ends = jax.vmap(lambda rs: jnp.searchsorted(rs, rv + WH))(r_s)
  se = jnp.stack([starts, ends], axis=1).astype(jnp.int32)
  se = jnp.pad(se, ((0, 0), (0, 0), (0, 16))).reshape(-1)
  packed = jnp.pad(packed, ((0, 0), (0, 16))).reshape(-1)

  canvas, counts = _sc_scatter(patch_logits, packed, se)
  canvas = canvas.reshape(B, C, H, W)
  counts = counts.reshape(B, H, W)
  return _normalize(canvas, counts, prev_pred.astype(jnp.float32))


# speedup vs baseline: 4.3134x; 1.6436x over previous
import functools

import jax
import jax.numpy as jnp
from jax import lax
from jax.experimental import pallas as pl
from jax.experimental.pallas import tpu as pltpu
from jax.experimental.pallas import tpu_sc as plsc

MIN_COV = 1e-6

B, K, C, PS = 2, 1024, 32, 16
H, W = 512, 512
WH = 16
CG = 4
NCG = C // CG
NYW = H // WH
NROUNDS = B * NCG
KP = K + 16
SEW = NYW + 16


def _sc_scatter(patch_logits, packed, se):
  mesh = plsc.VectorSubcoreMesh(core_axis_name="c", subcore_axis_name="s")

  @functools.partial(
      pl.kernel,
      out_type=(
          jax.ShapeDtypeStruct((B, C, H * W), jnp.float32),
          jax.ShapeDtypeStruct((B, H * W), jnp.float32),
      ),
      mesh=mesh,
      compiler_params=pltpu.CompilerParams(needs_layout_passes=False),
      scratch_types=[
          pltpu.VMEM((B * KP,), jnp.int32),
          pltpu.VMEM((B * 2 * SEW,), jnp.int32),
          pltpu.VMEM((CG * WH * W,), jnp.float32),
          pltpu.VMEM((WH * W,), jnp.float32),
          [pltpu.VMEM((CG, PS, PS), jnp.float32)] * 4,
          [pltpu.SemaphoreType.DMA] * 4,
          pltpu.SemaphoreType.DMA,
      ],
  )
  def scatter_kernel(patch_hbm, packed_hbm, se_hbm, out_hbm, cnt_hbm,
                     pk_v, se_v, canvas, cntw, bufs, sems, fsem):
    cid = lax.axis_index("c")
    sid = lax.axis_index("s")
    wid = sid * 2 + cid
    rbase = wid * WH

    pltpu.sync_copy(packed_hbm, pk_v)
    pltpu.sync_copy(se_hbm, se_v)

    zeros16 = jnp.zeros((16,), jnp.float32)
    ones16 = jnp.ones((16,), jnp.float32)
    iota16 = lax.iota(jnp.int32, 16)

    def _scalar_at(ref, flat_idx):
      return plsc.load_gather(ref, [jnp.full((16,), flat_idx, jnp.int32)])[0]

    def round_body(t, _):
      b = t // NCG
      cg = lax.rem(t, NCG)

      s = _scalar_at(se_v, (b * 2 + 0) * SEW + wid)
      e = _scalar_at(se_v, (b * 2 + 1) * SEW + wid)

      def fetch(i, buf, sem):
        p = _scalar_at(pk_v, b * KP + i)
        k = lax.shift_right_logical(p, 18)
        pltpu.async_copy(patch_hbm.at[b, k, pl.ds(cg * CG, CG)], buf, sem)

      def wait_buf(buf, sem):
        pltpu.make_async_copy(patch_hbm.at[0, 0, pl.ds(0, CG)], buf,
                              sem).wait()

      def scat(i, buf):
        p = _scalar_at(pk_v, b * KP + i)
        r = lax.shift_right_logical(p, 9) & 511
        cc = p & 511
        xidx = cc + iota16
        for dy in range(PS):
          yl = r + dy - rbase
          ok = (yl >= 0) & (yl < WH)

          @pl.when(ok)
          def _add():
            idx0 = yl * W + xidx
            for ch in range(CG):
              plsc.addupdate_scatter(canvas, [idx0 + ch * (WH * W)],
                                     buf[ch, dy, :])

            @pl.when(cg == 0)
            def _cnt():
              plsc.addupdate_scatter(cntw, [idx0], ones16)

      for j in range(4):
        @pl.when(s + j < e)
        def _prime(j=j):
          fetch(s + j, bufs[j], sems[j])

      def zrow(q, _):
        base = pl.multiple_of(q * 256, 256)
        for j in range(16):
          canvas[pl.ds(base + j * 16, 16)] = zeros16
        return 0
      lax.fori_loop(0, CG * WH * W // 256, zrow, 0)

      def zcnt(q, _):
        base = pl.multiple_of(q * 256, 256)
        for j in range(16):
          cntw[pl.ds(base + j * 16, 16)] = zeros16
        return 0
      lax.fori_loop(0, WH * W // 256, zcnt, 0)

      def pgroup(q, _):
        i = s + 4 * q
        for j in range(4):
          @pl.when(i + j < e)
          def _one(j=j):
            wait_buf(bufs[j], sems[j])
            scat(i + j, bufs[j])

            @pl.when(i + j + 4 < e)
            def _refill():
              fetch(i + j + 4, bufs[j], sems[j])

        return 0

      lax.fori_loop(0, (e - s + 3) // 4, pgroup, 0)

      for ch in range(CG):
        pltpu.async_copy(
            canvas.at[pl.ds(ch * WH * W, WH * W)],
            out_hbm.at[b, cg * CG + ch, pl.ds(rbase * W, WH * W)], fsem)

      @pl.when(cg == 0)
      def _flush_cnt():
        pltpu.async_copy(cntw, cnt_hbm.at[b, pl.ds(rbase * W, WH * W)], fsem)

      for ch in range(CG):
        pltpu.make_async_copy(
            canvas.at[pl.ds(ch * WH * W, WH * W)],
            out_hbm.at[b, cg * CG + ch, pl.ds(rbase * W, WH * W)],
            fsem).wait()

      @pl.when(cg == 0)
      def _drain_cnt():
        pltpu.make_async_copy(cntw, cnt_hbm.at[b, pl.ds(rbase * W, WH * W)],
                              fsem).wait()

      return 0

    lax.fori_loop(0, NROUNDS, round_body, 0)

  return scatter_kernel(patch_logits, packed, se)


def _norm_body(can_ref, cnt_ref, prev_ref, out_ref):
  cnt = cnt_ref[...]
  covered = cnt > MIN_COV
  safe = jnp.maximum(cnt, MIN_COV)
  out_ref[...] = jnp.where(covered[:, None],
                           can_ref[...] / safe[:, None],
                           prev_ref[...])


def _normalize(canvas, counts, prev_pred):
  grid = (B, NCG)
  return pl.pallas_call(
      _norm_body,
      grid=grid,
      in_specs=[
          pl.BlockSpec((1, CG, H, W), lambda b, g: (b, g, 0, 0)),
          pl.BlockSpec((1, H, W), lambda b, g: (b, 0, 0)),
          pl.BlockSpec((1, CG, H, W), lambda b, g: (b, g, 0, 0)),
      ],
      out_specs=pl.BlockSpec((1, CG, H, W), lambda b, g: (b, g, 0, 0)),
      out_shape=jax.ShapeDtypeStruct((B, C, H, W), jnp.float32),
  )(canvas, counts, prev_pred)


def kernel(patch_logits, coords, output_size, prev_pred):
  del output_size
  r = coords[:, :, 0].astype(jnp.int32)
  cc = coords[:, :, 1].astype(jnp.int32)
  order = jnp.argsort(r, axis=1).astype(jnp.int32)
  r_s = jnp.take_along_axis(r, order, axis=1)
  c_s = jnp.take_along_axis(cc, order, axis=1)
  packed = (order << 18) | (r_s << 9) | c_s

  rv = jnp.arange(NYW, dtype=jnp.int32) * WH
  starts = jax.vmap(lambda rs: jnp.searchsorted(rs, rv - (PS - 1)))(r_s)
  ends = jax.vmap(lambda rs: jnp.searchsorted(rs, rv + WH))(r_s)
  se = jnp.stack([starts, ends], axis=1).astype(jnp.int32)
  se = jnp.pad(se, ((0, 0), (0, 0), (0, 16))).reshape(-1)
  packed = jnp.pad(packed, ((0, 0), (0, 16))).reshape(-1)

  canvas, counts = _sc_scatter(patch_logits, packed, se)
  canvas = canvas.reshape(B, C, H, W)
  counts = counts.reshape(B, H, W)
  return _normalize(canvas, counts, prev_pred.astype(jnp.float32))
